# Initial kernel scaffold; baseline (speedup 1.0000x reference)
import jax
import jax.numpy as jnp
from jax.experimental import pallas as pl


def kernel(feature, edge_index, W1, b1, W2, b2, Wp, bp):
    raise NotImplementedError("write your pallas kernel here")


# speedup vs baseline: 3.0801x; 3.0801x over previous
import jax
import jax.numpy as jnp
from jax import lax
from jax.experimental import pallas as pl
from jax.experimental.pallas import tpu as pltpu
from jax.experimental.pallas import tpu_sc as plsc

N = 10000
E = 320000
D = 128
NCLS = 40

NC = 2
NS = 16
NW = NC * NS
CH = 128
EPAD = NW * CH * 80
ROWS2D = EPAD // CH
NCHUNK = ROWS2D // NW
NPAD = 10240
RZ = NPAD // NS
ZR = 32

_mesh = plsc.VectorSubcoreMesh(
    core_axis_name="c", subcore_axis_name="s", num_cores=NC, num_subcores=NS
)


def _seg_sum_body(table, src2d, dst2d, out, sidx, didx, rows, zbuf, acc, sem):
    c = lax.axis_index("c")
    s = lax.axis_index("s")
    wid = s * NC + c

    def _zrow(i, carry):
        def _zcol(j, carry2):
            zbuf[i, pl.ds(j * 16, 16)] = jnp.zeros((16,), jnp.float32)
            return carry2
        return lax.fori_loop(0, D // 16, _zcol, carry)
    lax.fori_loop(0, ZR, _zrow, 0)

    def _zslice(k, carry):
        pltpu.sync_copy(zbuf, acc.at[pl.ds(s * RZ + k * ZR, ZR)])
        return carry
    lax.fori_loop(0, RZ // ZR, _zslice, 0)
    plsc.subcore_barrier()

    pltpu.sync_copy(src2d.at[pl.ds(wid * NCHUNK, NCHUNK)], sidx)
    pltpu.sync_copy(dst2d.at[pl.ds(wid * NCHUNK, NCHUNK)], didx)

    def _chunk(j, carry):
        pltpu.async_copy(table.at[sidx.at[j]], rows, sem).wait()
        pltpu.sync_copy(rows, acc.at[didx.at[j]], add=True)
        return carry
    lax.fori_loop(0, NCHUNK, _chunk, 0)
    plsc.subcore_barrier()

    pltpu.sync_copy(acc.at[pl.ds(s * RZ, RZ)], out.at[c, pl.ds(s * RZ, RZ)])


_seg_sum = pl.kernel(
    _seg_sum_body,
    out_type=jax.ShapeDtypeStruct((NC, NPAD, D), jnp.float32),
    mesh=_mesh,
    scratch_types=[
        pltpu.VMEM((NCHUNK, CH), jnp.int32),
        pltpu.VMEM((NCHUNK, CH), jnp.int32),
        pltpu.VMEM((CH, D), jnp.float32),
        pltpu.VMEM((ZR, D), jnp.float32),
        pltpu.VMEM_SHARED((NPAD, D), jnp.float32),
        pltpu.SemaphoreType.DMA,
    ],
)

BLK1 = 1024
BLK2 = 1000


def _lin_relu_body(p0, p1, w, b, out):
    a = p0[...] + p1[...]
    h = jnp.dot(a, w[...], preferred_element_type=jnp.float32) + b[...]
    out[...] = jnp.maximum(h, 0.0)


def _lin_relu(p0, p1, w, b):
    return pl.pallas_call(
        _lin_relu_body,
        grid=(NPAD // BLK1,),
        in_specs=[
            pl.BlockSpec((BLK1, D), lambda i: (i, 0)),
            pl.BlockSpec((BLK1, D), lambda i: (i, 0)),
            pl.BlockSpec((D, D), lambda i: (0, 0)),
            pl.BlockSpec((1, D), lambda i: (0, 0)),
        ],
        out_specs=pl.BlockSpec((BLK1, D), lambda i: (i, 0)),
        out_shape=jax.ShapeDtypeStruct((NPAD, D), jnp.float32),
    )(p0, p1, w, b)


def _final_body(q0, q1, w, b, wp, bp, out, acc):
    i = pl.program_id(0)
    a = q0[...] + q1[...]
    h = jnp.maximum(jnp.dot(a, w[...], preferred_element_type=jnp.float32) + b[...], 0.0)
    colsum = jnp.sum(h, axis=0, keepdims=True)

    @pl.when(i == 0)
    def _():
        acc[...] = colsum

    @pl.when(i > 0)
    def _():
        acc[...] = acc[...] + colsum

    @pl.when(i == N // BLK2 - 1)
    def _():
        g = acc[...] * (1.0 / N)
        out[...] = jnp.dot(g, wp[...], preferred_element_type=jnp.float32) + bp[...]


def _final(q0, q1, w, b, wp, bp):
    return pl.pallas_call(
        _final_body,
        grid=(N // BLK2,),
        in_specs=[
            pl.BlockSpec((BLK2, D), lambda i: (i, 0)),
            pl.BlockSpec((BLK2, D), lambda i: (i, 0)),
            pl.BlockSpec((D, D), lambda i: (0, 0)),
            pl.BlockSpec((1, D), lambda i: (0, 0)),
            pl.BlockSpec((D, D), lambda i: (0, 0)),
            pl.BlockSpec((1, D), lambda i: (0, 0)),
        ],
        out_specs=pl.BlockSpec((1, D), lambda i: (0, 0)),
        out_shape=jax.ShapeDtypeStruct((1, D), jnp.float32),
        scratch_shapes=[pltpu.VMEM((1, D), jnp.float32)],
    )(q0, q1, w, b, wp, bp)


def kernel(feature, edge_index, W1, b1, W2, b2, Wp, bp):
    npad = EPAD - E
    src2d = jnp.concatenate(
        [edge_index[0], jnp.zeros((npad,), jnp.int32)]).reshape(ROWS2D, CH)
    dst2d = jnp.concatenate(
        [edge_index[1], jnp.full((npad,), N, jnp.int32)]).reshape(ROWS2D, CH)

    p = _seg_sum(feature, src2d, dst2d)
    h1 = _lin_relu(p[0], p[1], W1, b1.reshape(1, D))
    q = _seg_sum(h1, src2d, dst2d)

    wp_pad = jnp.zeros((D, D), jnp.float32).at[:, :NCLS].set(Wp)
    bp_pad = jnp.zeros((1, D), jnp.float32).at[:, :NCLS].set(bp)
    out = _final(q[0], q[1], W2, b2.reshape(1, D), wp_pad, bp_pad)
    return out[:, :NCLS]


# speedup vs baseline: 3.4111x; 1.1075x over previous
import jax
import jax.numpy as jnp
from jax import lax
from jax.experimental import pallas as pl
from jax.experimental.pallas import tpu as pltpu
from jax.experimental.pallas import tpu_sc as plsc

N = 10000
E = 320000
D = 128
NCLS = 40

NC = 2
NS = 16
NW = NC * NS
CH = 128
EPAD = NW * CH * 80
ROWS2D = EPAD // CH
NCHUNK = ROWS2D // NW
NPAD = 10240
RZ = NPAD // NS
DBLK = 16

_mesh = plsc.VectorSubcoreMesh(
    core_axis_name="c", subcore_axis_name="s", num_cores=NC, num_subcores=NS
)


def _seg_sum_body(table, src2d, dst2d, out, sidx, didx, rows_a, rows_b,
                  acc, sem_a, sem_b):
    c = lax.axis_index("c")
    s = lax.axis_index("s")
    wid = s * NC + c

    def _zrow(i, carry):
        def _zcol(j, carry2):
            rows_a[i, pl.ds(j * 16, 16)] = jnp.zeros((16,), jnp.float32)
            return carry2
        return lax.fori_loop(0, D // 16, _zcol, carry)
    lax.fori_loop(0, CH, _zrow, 0)

    def _zslice(k, carry):
        pltpu.sync_copy(rows_a, acc.at[pl.ds(s * RZ + k * CH, CH)])
        return carry
    lax.fori_loop(0, RZ // CH, _zslice, 0)
    plsc.subcore_barrier()

    pltpu.sync_copy(src2d.at[pl.ds(wid * NCHUNK, NCHUNK)], sidx)

    def _wait(buf, s_):
        pltpu.make_async_copy(table.at[sidx.at[0]], buf, s_).wait()

    pltpu.async_copy(table.at[sidx.at[0]], rows_a, sem_a)

    def _blk(b, carry):
        pltpu.sync_copy(dst2d.at[pl.ds(wid * NCHUNK + b * DBLK, DBLK)], didx)

        def _chunk(jj, carry2):
            j = b * DBLK + 2 * jj
            jl = 2 * jj
            pltpu.async_copy(table.at[sidx.at[j + 1]], rows_b, sem_b)
            _wait(rows_a, sem_a)
            pltpu.sync_copy(rows_a, acc.at[didx.at[jl]], add=True)
            jn = lax.rem(j + 2, NCHUNK)
            pltpu.async_copy(table.at[sidx.at[jn]], rows_a, sem_a)
            _wait(rows_b, sem_b)
            pltpu.sync_copy(rows_b, acc.at[didx.at[jl + 1]], add=True)
            return carry2
        return lax.fori_loop(0, DBLK // 2, _chunk, carry)
    lax.fori_loop(0, NCHUNK // DBLK, _blk, 0)
    _wait(rows_a, sem_a)
    plsc.subcore_barrier()

    pltpu.sync_copy(acc.at[pl.ds(s * RZ, RZ)], out.at[c, pl.ds(s * RZ, RZ)])


_seg_sum = pl.kernel(
    _seg_sum_body,
    out_type=jax.ShapeDtypeStruct((NC, NPAD, D), jnp.float32),
    mesh=_mesh,
    scratch_types=[
        pltpu.VMEM((NCHUNK, CH), jnp.int32),
        pltpu.VMEM((DBLK, CH), jnp.int32),
        pltpu.VMEM((CH, D), jnp.float32),
        pltpu.VMEM((CH, D), jnp.float32),
        pltpu.VMEM_SHARED((NPAD, D), jnp.float32),
        pltpu.SemaphoreType.DMA,
        pltpu.SemaphoreType.DMA,
    ],
)

BLK1 = 1024
BLK2 = 1000


def _lin_relu_body(p0, p1, w, b, out):
    a = p0[...] + p1[...]
    h = jnp.dot(a, w[...], preferred_element_type=jnp.float32) + b[...]
    out[...] = jnp.maximum(h, 0.0)


def _lin_relu(p0, p1, w, b):
    return pl.pallas_call(
        _lin_relu_body,
        grid=(NPAD // BLK1,),
        in_specs=[
            pl.BlockSpec((BLK1, D), lambda i: (i, 0)),
            pl.BlockSpec((BLK1, D), lambda i: (i, 0)),
            pl.BlockSpec((D, D), lambda i: (0, 0)),
            pl.BlockSpec((1, D), lambda i: (0, 0)),
        ],
        out_specs=pl.BlockSpec((BLK1, D), lambda i: (i, 0)),
        out_shape=jax.ShapeDtypeStruct((NPAD, D), jnp.float32),
    )(p0, p1, w, b)


def _final_body(q0, q1, w, b, wp, bp, out, acc):
    i = pl.program_id(0)
    a = q0[...] + q1[...]
    h = jnp.maximum(jnp.dot(a, w[...], preferred_element_type=jnp.float32) + b[...], 0.0)
    colsum = jnp.sum(h, axis=0, keepdims=True)

    @pl.when(i == 0)
    def _():
        acc[...] = colsum

    @pl.when(i > 0)
    def _():
        acc[...] = acc[...] + colsum

    @pl.when(i == N // BLK2 - 1)
    def _():
        g = acc[...] * (1.0 / N)
        out[...] = jnp.dot(g, wp[...], preferred_element_type=jnp.float32) + bp[...]


def _final(q0, q1, w, b, wp, bp):
    return pl.pallas_call(
        _final_body,
        grid=(N // BLK2,),
        in_specs=[
            pl.BlockSpec((BLK2, D), lambda i: (i, 0)),
            pl.BlockSpec((BLK2, D), lambda i: (i, 0)),
            pl.BlockSpec((D, D), lambda i: (0, 0)),
            pl.BlockSpec((1, D), lambda i: (0, 0)),
            pl.BlockSpec((D, D), lambda i: (0, 0)),
            pl.BlockSpec((1, D), lambda i: (0, 0)),
        ],
        out_specs=pl.BlockSpec((1, D), lambda i: (0, 0)),
        out_shape=jax.ShapeDtypeStruct((1, D), jnp.float32),
        scratch_shapes=[pltpu.VMEM((1, D), jnp.float32)],
    )(q0, q1, w, b, wp, bp)


def kernel(feature, edge_index, W1, b1, W2, b2, Wp, bp):
    npad = EPAD - E
    src2d = jnp.concatenate(
        [edge_index[0], jnp.zeros((npad,), jnp.int32)]).reshape(ROWS2D, CH)
    pad_dst = N + jnp.arange(npad, dtype=jnp.int32) % (NPAD - N)
    dst2d = jnp.concatenate([edge_index[1], pad_dst]).reshape(ROWS2D, CH)

    p = _seg_sum(feature, src2d, dst2d)
    h1 = _lin_relu(p[0], p[1], W1, b1.reshape(1, D))
    q = _seg_sum(h1, src2d, dst2d)

    wp_pad = jnp.zeros((D, D), jnp.float32).at[:, :NCLS].set(Wp)
    bp_pad = jnp.zeros((1, D), jnp.float32).at[:, :NCLS].set(bp)
    out = _final(q[0], q[1], W2, b2.reshape(1, D), wp_pad, bp_pad)
    return out[:, :NCLS]


# speedup vs baseline: 12.3394x; 3.6174x over previous
import jax
import jax.numpy as jnp
from jax import lax
from jax.experimental import pallas as pl
from jax.experimental.pallas import tpu as pltpu
from jax.experimental.pallas import tpu_sc as plsc

N = 10000
E = 320000
D = 128
NCLS = 40

NC = 2
NS = 16
NW = NC * NS
CH = 128
EPAD = NW * CH * 80
ROWS2D = EPAD // CH
NCHUNK = ROWS2D // NW
NPAD = 10240
RZ = NPAD // NS
DBLK = 16

_mesh = plsc.VectorSubcoreMesh(
    core_axis_name="c", subcore_axis_name="s", num_cores=NC, num_subcores=NS
)


def _seg_sum_body(table, src2d, dst2d, out, sidx, didx, rows_a, rows_b,
                  acc, sem_a, sem_b):
    c = lax.axis_index("c")
    s = lax.axis_index("s")
    wid = s * NC + c

    def _zrow(i, carry):
        def _zcol(j, carry2):
            rows_a[i, pl.ds(j * 16, 16)] = jnp.zeros((16,), jnp.float32)
            return carry2
        return lax.fori_loop(0, D // 16, _zcol, carry)
    lax.fori_loop(0, CH, _zrow, 0)

    def _zslice(k, carry):
        pltpu.sync_copy(rows_a, acc.at[pl.ds(s * RZ + k * CH, CH)])
        return carry
    lax.fori_loop(0, RZ // CH, _zslice, 0)
    plsc.subcore_barrier()

    pltpu.sync_copy(src2d.at[pl.ds(wid * NCHUNK, NCHUNK)], sidx)

    def _wait(buf, s_):
        pltpu.make_async_copy(table.at[sidx.at[0]], buf, s_).wait()

    pltpu.async_copy(table.at[sidx.at[0]], rows_a, sem_a)

    def _blk(b, carry):
        pltpu.sync_copy(dst2d.at[pl.ds(wid * NCHUNK + b * DBLK, DBLK)], didx)

        def _chunk(jj, carry2):
            j = b * DBLK + 2 * jj
            jl = 2 * jj
            pltpu.async_copy(table.at[sidx.at[j + 1]], rows_b, sem_b)
            _wait(rows_a, sem_a)
            pltpu.sync_copy(rows_a, acc.at[didx.at[jl]], add=True)
            jn = lax.rem(j + 2, NCHUNK)
            pltpu.async_copy(table.at[sidx.at[jn]], rows_a, sem_a)
            _wait(rows_b, sem_b)
            pltpu.sync_copy(rows_b, acc.at[didx.at[jl + 1]], add=True)
            return carry2
        return lax.fori_loop(0, DBLK // 2, _chunk, carry)
    lax.fori_loop(0, NCHUNK // DBLK, _blk, 0)
    _wait(rows_a, sem_a)
    plsc.subcore_barrier()

    pltpu.sync_copy(acc.at[pl.ds(s * RZ, RZ)], out.at[c, pl.ds(s * RZ, RZ)])


_seg_sum = pl.kernel(
    _seg_sum_body,
    out_type=jax.ShapeDtypeStruct((NC, NPAD, D), jnp.float32),
    mesh=_mesh,
    scratch_types=[
        pltpu.VMEM((NCHUNK, CH), jnp.int32),
        pltpu.VMEM((DBLK, CH), jnp.int32),
        pltpu.VMEM((CH, D), jnp.float32),
        pltpu.VMEM((CH, D), jnp.float32),
        pltpu.VMEM_SHARED((NPAD, D), jnp.float32),
        pltpu.SemaphoreType.DMA,
        pltpu.SemaphoreType.DMA,
    ],
)

BLK1 = 1024
BLK2 = 1000


def _lin_relu_body(p0, p1, w, b, out):
    a = p0[...] + p1[...]
    h = jnp.dot(a, w[...], preferred_element_type=jnp.float32) + b[...]
    out[...] = jnp.maximum(h, 0.0)


def _lin_relu(p0, p1, w, b):
    return pl.pallas_call(
        _lin_relu_body,
        grid=(NPAD // BLK1,),
        in_specs=[
            pl.BlockSpec((BLK1, D), lambda i: (i, 0)),
            pl.BlockSpec((BLK1, D), lambda i: (i, 0)),
            pl.BlockSpec((D, D), lambda i: (0, 0)),
            pl.BlockSpec((1, D), lambda i: (0, 0)),
        ],
        out_specs=pl.BlockSpec((BLK1, D), lambda i: (i, 0)),
        out_shape=jax.ShapeDtypeStruct((NPAD, D), jnp.float32),
    )(p0, p1, w, b)


def _final_body(q0, q1, w, b, wp, bp, out, acc):
    i = pl.program_id(0)
    a = q0[...] + q1[...]
    h = jnp.maximum(jnp.dot(a, w[...], preferred_element_type=jnp.float32) + b[...], 0.0)
    colsum = jnp.sum(h, axis=0, keepdims=True)

    @pl.when(i == 0)
    def _():
        acc[...] = colsum

    @pl.when(i > 0)
    def _():
        acc[...] = acc[...] + colsum

    @pl.when(i == N // BLK2 - 1)
    def _():
        g = acc[...] * (1.0 / N)
        out[...] = jnp.dot(g, wp[...], preferred_element_type=jnp.float32) + bp[...]


def _final(q0, q1, w, b, wp, bp):
    return pl.pallas_call(
        _final_body,
        grid=(N // BLK2,),
        in_specs=[
            pl.BlockSpec((BLK2, D), lambda i: (i, 0)),
            pl.BlockSpec((BLK2, D), lambda i: (i, 0)),
            pl.BlockSpec((D, D), lambda i: (0, 0)),
            pl.BlockSpec((1, D), lambda i: (0, 0)),
            pl.BlockSpec((D, D), lambda i: (0, 0)),
            pl.BlockSpec((1, D), lambda i: (0, 0)),
        ],
        out_specs=pl.BlockSpec((1, D), lambda i: (0, 0)),
        out_shape=jax.ShapeDtypeStruct((1, D), jnp.float32),
        scratch_shapes=[pltpu.VMEM((1, D), jnp.float32)],
    )(q0, q1, w, b, wp, bp)


def kernel(feature, edge_index, W1, b1, W2, b2, Wp, bp):
    npad = EPAD - E
    pad_src = jnp.arange(npad, dtype=jnp.int32) % N
    src2d = jnp.concatenate([edge_index[0], pad_src]).reshape(ROWS2D, CH)
    pad_dst = N + jnp.arange(npad, dtype=jnp.int32) % (NPAD - N)
    dst2d = jnp.concatenate([edge_index[1], pad_dst]).reshape(ROWS2D, CH)

    p = _seg_sum(feature, src2d, dst2d)
    h1 = _lin_relu(p[0], p[1], W1, b1.reshape(1, D))
    q = _seg_sum(h1, src2d, dst2d)

    wp_pad = jnp.zeros((D, D), jnp.float32).at[:, :NCLS].set(Wp)
    bp_pad = jnp.zeros((1, D), jnp.float32).at[:, :NCLS].set(bp)
    out = _final(q[0], q[1], W2, b2.reshape(1, D), wp_pad, bp_pad)
    return out[:, :NCLS]


# speedup vs baseline: 13.0187x; 1.0550x over previous
import jax
import jax.numpy as jnp
from jax import lax
from jax.experimental import pallas as pl
from jax.experimental.pallas import tpu as pltpu
from jax.experimental.pallas import tpu_sc as plsc

N = 10000
E = 320000
D = 128
NCLS = 40

NC = 2
NS = 16
NW = NC * NS
CH = 128
EPAD = NW * CH * 80
ROWS2D = EPAD // CH
NCHUNK = ROWS2D // NW
NPAD = 10240
RZ = NPAD // NS
DBLK = 16

_mesh = plsc.VectorSubcoreMesh(
    core_axis_name="c", subcore_axis_name="s", num_cores=NC, num_subcores=NS
)


def _seg_sum_body(table, src2d, dst2d, out, sidx, didx, rows_a, rows_b,
                  acc, sem_a, sem_b):
    c = lax.axis_index("c")
    s = lax.axis_index("s")
    wid = s * NC + c

    def _zrow(i, carry):
        def _zcol(j, carry2):
            rows_a[i, pl.ds(j * 16, 16)] = jnp.zeros((16,), jnp.float32)
            return carry2
        return lax.fori_loop(0, D // 16, _zcol, carry)
    lax.fori_loop(0, CH, _zrow, 0)

    def _zslice(k, carry):
        pltpu.sync_copy(rows_a, acc.at[pl.ds(s * RZ + k * CH, CH)])
        return carry
    lax.fori_loop(0, RZ // CH, _zslice, 0)
    plsc.subcore_barrier()

    pltpu.sync_copy(src2d.at[pl.ds(wid * NCHUNK, NCHUNK)], sidx)

    def _wait(buf, s_):
        pltpu.make_async_copy(table.at[sidx.at[0]], buf, s_).wait()

    def _gather(j, buf, s_):
        pltpu.async_copy(table.at[sidx.at[j, pl.ds(0, CH // 2)]],
                         buf.at[pl.ds(0, CH // 2)], s_)
        pltpu.async_copy(table.at[sidx.at[j, pl.ds(CH // 2, CH // 2)]],
                         buf.at[pl.ds(CH // 2, CH // 2)], s_)

    _gather(0, rows_a, sem_a)

    def _blk(b, carry):
        pltpu.sync_copy(dst2d.at[pl.ds(wid * NCHUNK + b * DBLK, DBLK)], didx)

        def _chunk(jj, carry2):
            j = b * DBLK + 2 * jj
            jl = 2 * jj
            _gather(j + 1, rows_b, sem_b)
            _wait(rows_a, sem_a)
            pltpu.sync_copy(rows_a, acc.at[didx.at[jl]], add=True)
            jn = lax.rem(j + 2, NCHUNK)
            _gather(jn, rows_a, sem_a)
            _wait(rows_b, sem_b)
            pltpu.sync_copy(rows_b, acc.at[didx.at[jl + 1]], add=True)
            return carry2
        return lax.fori_loop(0, DBLK // 2, _chunk, carry)
    lax.fori_loop(0, NCHUNK // DBLK, _blk, 0)
    _wait(rows_a, sem_a)
    plsc.subcore_barrier()

    pltpu.sync_copy(acc.at[pl.ds(s * RZ, RZ)], out.at[c, pl.ds(s * RZ, RZ)])


_seg_sum = pl.kernel(
    _seg_sum_body,
    out_type=jax.ShapeDtypeStruct((NC, NPAD, D), jnp.float32),
    mesh=_mesh,
    scratch_types=[
        pltpu.VMEM((NCHUNK, CH), jnp.int32),
        pltpu.VMEM((DBLK, CH), jnp.int32),
        pltpu.VMEM((CH, D), jnp.float32),
        pltpu.VMEM((CH, D), jnp.float32),
        pltpu.VMEM_SHARED((NPAD, D), jnp.float32),
        pltpu.SemaphoreType.DMA,
        pltpu.SemaphoreType.DMA,
    ],
)

BLK1 = 1024
BLK2 = 1000


def _lin_relu_body(p, w, b, out):
    a = p[0] + p[1]
    h = jnp.dot(a, w[...], preferred_element_type=jnp.float32) + b[...]
    out[...] = jnp.maximum(h, 0.0)


def _lin_relu(p, w, b):
    return pl.pallas_call(
        _lin_relu_body,
        grid=(NPAD // BLK1,),
        in_specs=[
            pl.BlockSpec((NC, BLK1, D), lambda i: (0, i, 0)),
            pl.BlockSpec((D, D), lambda i: (0, 0)),
            pl.BlockSpec((1, D), lambda i: (0, 0)),
        ],
        out_specs=pl.BlockSpec((BLK1, D), lambda i: (i, 0)),
        out_shape=jax.ShapeDtypeStruct((NPAD, D), jnp.float32),
    )(p, w, b)


def _final_body(q, w, b, wp, bp, out, acc):
    i = pl.program_id(0)
    a = q[0] + q[1]
    h = jnp.maximum(jnp.dot(a, w[...], preferred_element_type=jnp.float32) + b[...], 0.0)
    colsum = jnp.sum(h, axis=0, keepdims=True)

    @pl.when(i == 0)
    def _():
        acc[...] = colsum

    @pl.when(i > 0)
    def _():
        acc[...] = acc[...] + colsum

    @pl.when(i == N // BLK2 - 1)
    def _():
        g = acc[...] * (1.0 / N)
        out[...] = jnp.dot(g, wp[...], preferred_element_type=jnp.float32) + bp[...]


def _final(q, w, b, wp, bp):
    return pl.pallas_call(
        _final_body,
        grid=(N // BLK2,),
        in_specs=[
            pl.BlockSpec((NC, BLK2, D), lambda i: (0, i, 0)),
            pl.BlockSpec((D, D), lambda i: (0, 0)),
            pl.BlockSpec((1, D), lambda i: (0, 0)),
            pl.BlockSpec((D, D), lambda i: (0, 0)),
            pl.BlockSpec((1, D), lambda i: (0, 0)),
        ],
        out_specs=pl.BlockSpec((1, D), lambda i: (0, 0)),
        out_shape=jax.ShapeDtypeStruct((1, D), jnp.float32),
        scratch_shapes=[pltpu.VMEM((1, D), jnp.float32)],
    )(q, w, b, wp, bp)


def kernel(feature, edge_index, W1, b1, W2, b2, Wp, bp):
    npad = EPAD - E
    pad_src = jnp.arange(npad, dtype=jnp.int32) % N
    src2d = jnp.concatenate([edge_index[0], pad_src]).reshape(ROWS2D, CH)
    pad_dst = N + jnp.arange(npad, dtype=jnp.int32) % (NPAD - N)
    dst2d = jnp.concatenate([edge_index[1], pad_dst]).reshape(ROWS2D, CH)

    p = _seg_sum(feature, src2d, dst2d)
    h1 = _lin_relu(p, W1, b1.reshape(1, D))
    q = _seg_sum(h1, src2d, dst2d)

    wp_pad = jnp.zeros((D, D), jnp.float32).at[:, :NCLS].set(Wp)
    bp_pad = jnp.zeros((1, D), jnp.float32).at[:, :NCLS].set(bp)
    out = _final(q, W2, b2.reshape(1, D), wp_pad, bp_pad)
    return out[:, :NCLS]


# speedup vs baseline: 13.2117x; 1.0148x over previous
import jax
import jax.numpy as jnp
from jax import lax
from jax.experimental import pallas as pl
from jax.experimental.pallas import tpu as pltpu
from jax.experimental.pallas import tpu_sc as plsc

N = 10000
E = 320000
D = 128
NCLS = 40

NC = 2
NS = 16
NW = NC * NS
CH = 128
EPAD = NW * CH * 80
ROWS2D = EPAD // CH
NCHUNK = ROWS2D // NW
NPAD = 10240
RZ = NPAD // NS
DBLK = 16

_mesh = plsc.VectorSubcoreMesh(
    core_axis_name="c", subcore_axis_name="s", num_cores=NC, num_subcores=NS
)


def _seg_sum_body(table, src2d, dst2d, out, sidx, didx0, didx1, rows_a,
                  rows_b, acc, sem_a, sem_b, sem_i, sem_d):
    c = lax.axis_index("c")
    s = lax.axis_index("s")
    wid = s * NC + c

    pltpu.async_copy(src2d.at[pl.ds(wid * NCHUNK, NCHUNK)], sidx, sem_i)
    pltpu.async_copy(dst2d.at[pl.ds(wid * NCHUNK, DBLK)], didx0, sem_d)

    def _zrow(i, carry):
        def _zcol(j, carry2):
            rows_a[i, pl.ds(j * 16, 16)] = jnp.zeros((16,), jnp.float32)
            return carry2
        return lax.fori_loop(0, D // 16, _zcol, carry)
    lax.fori_loop(0, CH, _zrow, 0)

    def _zslice(k, carry):
        pltpu.sync_copy(rows_a, acc.at[pl.ds(s * RZ + k * CH, CH)])
        return carry
    lax.fori_loop(0, RZ // CH, _zslice, 0)

    def _wait(buf, s_):
        pltpu.make_async_copy(table.at[sidx.at[0]], buf, s_).wait()

    def _wait_didx(buf):
        pltpu.make_async_copy(dst2d.at[pl.ds(0, DBLK)], buf, sem_d).wait()

    def _gather(j, buf, s_):
        pltpu.async_copy(table.at[sidx.at[j, pl.ds(0, CH // 2)]],
                         buf.at[pl.ds(0, CH // 2)], s_)
        pltpu.async_copy(table.at[sidx.at[j, pl.ds(CH // 2, CH // 2)]],
                         buf.at[pl.ds(CH // 2, CH // 2)], s_)

    pltpu.make_async_copy(src2d.at[pl.ds(0, NCHUNK)], sidx, sem_i).wait()
    _gather(0, rows_a, sem_a)
    plsc.subcore_barrier()

    dbufs = (didx0, didx1)
    for b in range(NCHUNK // DBLK):
        didx = dbufs[b % 2]
        _wait_didx(didx)
        if b + 1 < NCHUNK // DBLK:
            pltpu.async_copy(
                dst2d.at[pl.ds(wid * NCHUNK + (b + 1) * DBLK, DBLK)],
                dbufs[(b + 1) % 2], sem_d)

        def _chunk(jj, carry2, b=b, didx=didx):
            j = b * DBLK + 2 * jj
            jl = 2 * jj
            _gather(j + 1, rows_b, sem_b)
            _wait(rows_a, sem_a)
            pltpu.sync_copy(rows_a, acc.at[didx.at[jl]], add=True)
            jn = (j + 2) % NCHUNK if b + 1 == NCHUNK // DBLK else j + 2
            _gather(jn, rows_a, sem_a)
            _wait(rows_b, sem_b)
            pltpu.sync_copy(rows_b, acc.at[didx.at[jl + 1]], add=True)
            return carry2
        lax.fori_loop(0, DBLK // 2, _chunk, 0)
    _wait(rows_a, sem_a)
    plsc.subcore_barrier()

    pltpu.sync_copy(acc.at[pl.ds(s * RZ, RZ)], out.at[c, pl.ds(s * RZ, RZ)])


_seg_sum = pl.kernel(
    _seg_sum_body,
    out_type=jax.ShapeDtypeStruct((NC, NPAD, D), jnp.float32),
    mesh=_mesh,
    scratch_types=[
        pltpu.VMEM((NCHUNK, CH), jnp.int32),
        pltpu.VMEM((DBLK, CH), jnp.int32),
        pltpu.VMEM((DBLK, CH), jnp.int32),
        pltpu.VMEM((CH, D), jnp.float32),
        pltpu.VMEM((CH, D), jnp.float32),
        pltpu.VMEM_SHARED((NPAD, D), jnp.float32),
        pltpu.SemaphoreType.DMA,
        pltpu.SemaphoreType.DMA,
        pltpu.SemaphoreType.DMA,
        pltpu.SemaphoreType.DMA,
    ],
)

BLK1 = 1024
BLK2 = 1000


def _lin_relu_body(p, w, b, out):
    a = p[0] + p[1]
    h = jnp.dot(a, w[...], preferred_element_type=jnp.float32) + b[...]
    out[...] = jnp.maximum(h, 0.0)


def _lin_relu(p, w, b):
    return pl.pallas_call(
        _lin_relu_body,
        grid=(NPAD // BLK1,),
        in_specs=[
            pl.BlockSpec((NC, BLK1, D), lambda i: (0, i, 0)),
            pl.BlockSpec((D, D), lambda i: (0, 0)),
            pl.BlockSpec((1, D), lambda i: (0, 0)),
        ],
        out_specs=pl.BlockSpec((BLK1, D), lambda i: (i, 0)),
        out_shape=jax.ShapeDtypeStruct((NPAD, D), jnp.float32),
    )(p, w, b)


def _final_body(q, w, b, wp, bp, out, acc):
    i = pl.program_id(0)
    a = q[0] + q[1]
    h = jnp.maximum(jnp.dot(a, w[...], preferred_element_type=jnp.float32) + b[...], 0.0)
    colsum = jnp.sum(h, axis=0, keepdims=True)

    @pl.when(i == 0)
    def _():
        acc[...] = colsum

    @pl.when(i > 0)
    def _():
        acc[...] = acc[...] + colsum

    @pl.when(i == N // BLK2 - 1)
    def _():
        g = acc[...] * (1.0 / N)
        out[...] = jnp.dot(g, wp[...], preferred_element_type=jnp.float32) + bp[...]


def _final(q, w, b, wp, bp):
    return pl.pallas_call(
        _final_body,
        grid=(N // BLK2,),
        in_specs=[
            pl.BlockSpec((NC, BLK2, D), lambda i: (0, i, 0)),
            pl.BlockSpec((D, D), lambda i: (0, 0)),
            pl.BlockSpec((1, D), lambda i: (0, 0)),
            pl.BlockSpec((D, D), lambda i: (0, 0)),
            pl.BlockSpec((1, D), lambda i: (0, 0)),
        ],
        out_specs=pl.BlockSpec((1, D), lambda i: (0, 0)),
        out_shape=jax.ShapeDtypeStruct((1, D), jnp.float32),
        scratch_shapes=[pltpu.VMEM((1, D), jnp.float32)],
    )(q, w, b, wp, bp)


def kernel(feature, edge_index, W1, b1, W2, b2, Wp, bp):
    npad = EPAD - E
    pad_src = jnp.arange(npad, dtype=jnp.int32) % N
    src2d = jnp.concatenate([edge_index[0], pad_src]).reshape(ROWS2D, CH)
    pad_dst = N + jnp.arange(npad, dtype=jnp.int32) % (NPAD - N)
    dst2d = jnp.concatenate([edge_index[1], pad_dst]).reshape(ROWS2D, CH)

    p = _seg_sum(feature, src2d, dst2d)
    h1 = _lin_relu(p, W1, b1.reshape(1, D))
    q = _seg_sum(h1, src2d, dst2d)

    wp_pad = jnp.zeros((D, D), jnp.float32).at[:, :NCLS].set(Wp)
    bp_pad = jnp.zeros((1, D), jnp.float32).at[:, :NCLS].set(bp)
    out = _final(q, W2, b2.reshape(1, D), wp_pad, bp_pad)
    return out[:, :NCLS]


# speedup vs baseline: 13.2651x; 1.0040x over previous
import jax
import jax.numpy as jnp
from jax import lax
from jax.experimental import pallas as pl
from jax.experimental.pallas import tpu as pltpu
from jax.experimental.pallas import tpu_sc as plsc

N = 10000
E = 320000
D = 128
NCLS = 40

NC = 2
NS = 16
NW = NC * NS
CH = 128
EPAD = NW * CH * 80
ROWS2D = EPAD // CH
NCHUNK = ROWS2D // NW
NPAD = 10240
RZ = NPAD // NS
DBLK = 16

_mesh = plsc.VectorSubcoreMesh(
    core_axis_name="c", subcore_axis_name="s", num_cores=NC, num_subcores=NS
)


def _seg_sum_body(table, src2d, dst2d, out, sidx, didx0, didx1, rows_a,
                  rows_b, acc, sem_a, sem_b, sem_i, sem_d):
    c = lax.axis_index("c")
    s = lax.axis_index("s")
    wid = s * NC + c

    pltpu.async_copy(src2d.at[pl.ds(wid * NCHUNK, NCHUNK)], sidx, sem_i)
    pltpu.async_copy(dst2d.at[pl.ds(wid * NCHUNK, DBLK)], didx0, sem_d)

    def _zrow(i, carry):
        def _zcol(j, carry2):
            rows_a[i, pl.ds(j * 16, 16)] = jnp.zeros((16,), jnp.float32)
            return carry2
        return lax.fori_loop(0, D // 16, _zcol, carry)
    lax.fori_loop(0, CH, _zrow, 0)

    def _zslice(k, carry):
        pltpu.sync_copy(rows_a, acc.at[pl.ds(s * RZ + k * CH, CH)])
        return carry
    lax.fori_loop(0, RZ // CH, _zslice, 0)

    def _wait(buf, s_):
        pltpu.make_async_copy(table.at[sidx.at[0]], buf, s_).wait()

    def _wait_didx(buf):
        pltpu.make_async_copy(dst2d.at[pl.ds(0, DBLK)], buf, sem_d).wait()

    def _gather(j, buf, s_):
        pltpu.async_copy(table.at[sidx.at[j]], buf, s_)

    pltpu.make_async_copy(src2d.at[pl.ds(0, NCHUNK)], sidx, sem_i).wait()
    _gather(0, rows_a, sem_a)
    plsc.subcore_barrier()

    dbufs = (didx0, didx1)
    for b in range(NCHUNK // DBLK):
        didx = dbufs[b % 2]
        _wait_didx(didx)
        if b + 1 < NCHUNK // DBLK:
            pltpu.async_copy(
                dst2d.at[pl.ds(wid * NCHUNK + (b + 1) * DBLK, DBLK)],
                dbufs[(b + 1) % 2], sem_d)

        def _chunk(jj, carry2, b=b, didx=didx):
            j = b * DBLK + 2 * jj
            jl = 2 * jj
            _gather(j + 1, rows_b, sem_b)
            _wait(rows_a, sem_a)
            pltpu.sync_copy(rows_a, acc.at[didx.at[jl]], add=True)
            jn = (j + 2) % NCHUNK if b + 1 == NCHUNK // DBLK else j + 2
            _gather(jn, rows_a, sem_a)
            _wait(rows_b, sem_b)
            pltpu.sync_copy(rows_b, acc.at[didx.at[jl + 1]], add=True)
            return carry2
        lax.fori_loop(0, DBLK // 2, _chunk, 0)
    _wait(rows_a, sem_a)
    plsc.subcore_barrier()

    pltpu.sync_copy(acc.at[pl.ds(s * RZ, RZ)], out.at[c, pl.ds(s * RZ, RZ)])


_seg_sum = pl.kernel(
    _seg_sum_body,
    out_type=jax.ShapeDtypeStruct((NC, NPAD, D), jnp.float32),
    mesh=_mesh,
    scratch_types=[
        pltpu.VMEM((NCHUNK, CH), jnp.int32),
        pltpu.VMEM((DBLK, CH), jnp.int32),
        pltpu.VMEM((DBLK, CH), jnp.int32),
        pltpu.VMEM((CH, D), jnp.float32),
        pltpu.VMEM((CH, D), jnp.float32),
        pltpu.VMEM_SHARED((NPAD, D), jnp.float32),
        pltpu.SemaphoreType.DMA,
        pltpu.SemaphoreType.DMA,
        pltpu.SemaphoreType.DMA,
        pltpu.SemaphoreType.DMA,
    ],
)

BLK1 = 1024
BLK2 = 1000


def _lin_relu_body(p, w, b, out):
    a = p[0] + p[1]
    h = jnp.dot(a, w[...], preferred_element_type=jnp.float32) + b[...]
    out[...] = jnp.maximum(h, 0.0)


def _lin_relu(p, w, b):
    return pl.pallas_call(
        _lin_relu_body,
        grid=(NPAD // BLK1,),
        in_specs=[
            pl.BlockSpec((NC, BLK1, D), lambda i: (0, i, 0)),
            pl.BlockSpec((D, D), lambda i: (0, 0)),
            pl.BlockSpec((1, D), lambda i: (0, 0)),
        ],
        out_specs=pl.BlockSpec((BLK1, D), lambda i: (i, 0)),
        out_shape=jax.ShapeDtypeStruct((NPAD, D), jnp.float32),
    )(p, w, b)


def _final_body(q, w, b, wp, bp, out, acc):
    i = pl.program_id(0)
    a = q[0] + q[1]
    h = jnp.maximum(jnp.dot(a, w[...], preferred_element_type=jnp.float32) + b[...], 0.0)
    colsum = jnp.sum(h, axis=0, keepdims=True)

    @pl.when(i == 0)
    def _():
        acc[...] = colsum

    @pl.when(i > 0)
    def _():
        acc[...] = acc[...] + colsum

    @pl.when(i == N // BLK2 - 1)
    def _():
        g = acc[...] * (1.0 / N)
        out[...] = jnp.dot(g, wp[...], preferred_element_type=jnp.float32) + bp[...]


def _final(q, w, b, wp, bp):
    return pl.pallas_call(
        _final_body,
        grid=(N // BLK2,),
        in_specs=[
            pl.BlockSpec((NC, BLK2, D), lambda i: (0, i, 0)),
            pl.BlockSpec((D, D), lambda i: (0, 0)),
            pl.BlockSpec((1, D), lambda i: (0, 0)),
            pl.BlockSpec((D, D), lambda i: (0, 0)),
            pl.BlockSpec((1, D), lambda i: (0, 0)),
        ],
        out_specs=pl.BlockSpec((1, D), lambda i: (0, 0)),
        out_shape=jax.ShapeDtypeStruct((1, D), jnp.float32),
        scratch_shapes=[pltpu.VMEM((1, D), jnp.float32)],
    )(q, w, b, wp, bp)


def kernel(feature, edge_index, W1, b1, W2, b2, Wp, bp):
    npad = EPAD - E
    pad_src = jnp.arange(npad, dtype=jnp.int32) % N
    src2d = jnp.concatenate([edge_index[0], pad_src]).reshape(ROWS2D, CH)
    pad_dst = N + jnp.arange(npad, dtype=jnp.int32) % (NPAD - N)
    dst2d = jnp.concatenate([edge_index[1], pad_dst]).reshape(ROWS2D, CH)

    p = _seg_sum(feature, src2d, dst2d)
    h1 = _lin_relu(p, W1, b1.reshape(1, D))
    q = _seg_sum(h1, src2d, dst2d)

    wp_pad = jnp.zeros((D, D), jnp.float32).at[:, :NCLS].set(Wp)
    bp_pad = jnp.zeros((1, D), jnp.float32).at[:, :NCLS].set(bp)
    out = _final(q, W2, b2.reshape(1, D), wp_pad, bp_pad)
    return out[:, :NCLS]
